# Initial kernel scaffold; baseline (speedup 1.0000x reference)
import jax
import jax.numpy as jnp
from jax.experimental import pallas as pl


def kernel(input, indices):
    raise NotImplementedError("write your pallas kernel here")


# speedup vs baseline: 3.8999x; 3.8999x over previous
import dataclasses
import functools

import jax
import jax.numpy as jnp
from jax import lax
from jax.experimental import pallas as pl
from jax.experimental.pallas import tpu as pltpu
from jax.experimental.pallas import tpu_sc as plsc

N, C = 4, 64
D_IN, H_IN, W_IN = 16, 28, 28
D_OUT, H_OUT, W_OUT = 32, 56, 56
L_IN = D_IN * H_IN * W_IN
L_OUT = D_OUT * H_OUT * W_OUT
ROWS = N * C
NUM_CORES = 2
NUM_SUBCORES = 16
NUM_WORKERS = NUM_CORES * NUM_SUBCORES
ROWS_PER_W = ROWS // NUM_WORKERS
LANES = 16
DUMP = L_OUT


def _sc_unpool(vals, scat):
    mesh = plsc.VectorSubcoreMesh(core_axis_name="c", subcore_axis_name="s")
    cp = pltpu.CompilerParams()
    if "needs_layout_passes" in pltpu.CompilerParams.__dataclass_fields__:
        cp = dataclasses.replace(cp, needs_layout_passes=False)

    @functools.partial(
        pl.kernel,
        compiler_params=cp,
        out_type=jax.ShapeDtypeStruct((ROWS, L_OUT), jnp.float32),
        mesh=mesh,
        scratch_types=[
            pltpu.VMEM((L_OUT + LANES,), jnp.float32),
            pltpu.VMEM((L_IN,), jnp.int32),
            pltpu.VMEM((L_IN,), jnp.float32),
            pltpu.SemaphoreType.DMA,
            pltpu.SemaphoreType.DMA,
        ],
    )
    def sc_kernel(val_hbm, scat_hbm, out_hbm, out_v, idx_v, val_v, sem_i, sem_v):
        wid = lax.axis_index("s") * NUM_CORES + lax.axis_index("c")
        zeros = jnp.zeros((LANES,), jnp.float32)

        @pl.loop(0, ROWS_PER_W)
        def _(k):
            row = wid * ROWS_PER_W + k
            cp_i = pltpu.async_copy(scat_hbm.at[row], idx_v, sem_i)
            cp_v = pltpu.async_copy(val_hbm.at[row], val_v, sem_v)

            @pl.loop(0, L_OUT + LANES, step=LANES)
            def _(i):
                out_v[pl.ds(i, LANES)] = zeros

            cp_i.wait()
            cp_v.wait()

            @pl.loop(0, L_IN, step=LANES)
            def _(i):
                iv = idx_v[pl.ds(i, LANES)]
                vv = val_v[pl.ds(i, LANES)]
                plsc.store_scatter(out_v, [iv], vv)

            pltpu.sync_copy(out_v.at[pl.ds(0, L_OUT)], out_hbm.at[row])

    return sc_kernel(vals, scat)


def kernel(input, indices):
    x = input.reshape(ROWS, L_IN)
    idx = indices.reshape(ROWS, L_IN).astype(jnp.int32)
    g = idx + jnp.arange(ROWS, dtype=jnp.int32)[:, None] * L_OUT
    ks, vs = lax.sort_key_val(g.reshape(-1), x.reshape(-1), is_stable=False)
    last = jnp.concatenate([ks[1:] != ks[:-1], jnp.ones((1,), jnp.bool_)])
    scat = jnp.where(last, ks % L_OUT, DUMP).astype(jnp.int32)
    out = _sc_unpool(vs.reshape(ROWS, L_IN), scat.reshape(ROWS, L_IN))
    return out.reshape(N, C, D_OUT, H_OUT, W_OUT)


# speedup vs baseline: 3.9652x; 1.0167x over previous
import dataclasses
import functools

import jax
import jax.numpy as jnp
from jax import lax
from jax.experimental import pallas as pl
from jax.experimental.pallas import tpu as pltpu
from jax.experimental.pallas import tpu_sc as plsc

N, C = 4, 64
D_IN, H_IN, W_IN = 16, 28, 28
D_OUT, H_OUT, W_OUT = 32, 56, 56
L_IN = D_IN * H_IN * W_IN
L_OUT = D_OUT * H_OUT * W_OUT
HALF = L_OUT // 2
ROWS = N * C
NUM_CORES = 2
NUM_SUBCORES = 16
NUM_WORKERS = NUM_CORES * NUM_SUBCORES
ROWS_PER_W = ROWS // NUM_WORKERS
LANES = 16
DUMP = L_OUT


def _sc_unpool(vals, scat):
    mesh = plsc.VectorSubcoreMesh(core_axis_name="c", subcore_axis_name="s")
    cp = pltpu.CompilerParams()
    if "needs_layout_passes" in pltpu.CompilerParams.__dataclass_fields__:
        cp = dataclasses.replace(cp, needs_layout_passes=False)

    @functools.partial(
        pl.kernel,
        compiler_params=cp,
        out_type=jax.ShapeDtypeStruct((ROWS, L_OUT), jnp.float32),
        mesh=mesh,
        scratch_types=[
            pltpu.VMEM((HALF,), jnp.float32),
            pltpu.VMEM((HALF,), jnp.float32),
            pltpu.VMEM((L_IN,), jnp.int32),
            pltpu.VMEM((L_IN,), jnp.float32),
            pltpu.SemaphoreType.DMA,
            pltpu.SemaphoreType.DMA,
            pltpu.SemaphoreType.DMA,
            pltpu.SemaphoreType.DMA,
        ],
    )
    def sc_kernel(val_hbm, scat_hbm, out_hbm, buf_a, buf_b, idx_v, val_v,
                  sem_i, sem_v, sem_oa, sem_ob):
        wid = lax.axis_index("s") * NUM_CORES + lax.axis_index("c")
        base = wid * ROWS_PER_W
        zeros = jnp.zeros((LANES,), jnp.float32)

        @pl.loop(0, HALF, step=LANES)
        def _(i):
            buf_a[pl.ds(i, LANES)] = zeros
            buf_b[pl.ds(i, LANES)] = zeros

        @pl.loop(0, ROWS_PER_W)
        def _(k):
            row = base + k

            @pl.when(k > 0)
            def _():
                prev = row - 1
                pltpu.make_async_copy(
                    buf_b, out_hbm.at[prev, pl.ds(HALF, HALF)], sem_ob
                ).wait()

                @pl.loop(0, L_IN, step=LANES)
                def _(i):
                    iv = idx_v[pl.ds(i, LANES)]
                    m = (iv >= HALF) & (iv < L_OUT)
                    plsc.store_scatter(
                        buf_b, [jnp.where(m, iv - HALF, 0)], zeros, mask=m)

                pltpu.make_async_copy(
                    buf_a, out_hbm.at[prev, pl.ds(0, HALF)], sem_oa
                ).wait()

                @pl.loop(0, L_IN, step=LANES)
                def _(i):
                    iv = idx_v[pl.ds(i, LANES)]
                    m = iv < HALF
                    plsc.store_scatter(buf_a, [iv], zeros, mask=m)

            cp_i = pltpu.async_copy(scat_hbm.at[row], idx_v, sem_i)
            cp_v = pltpu.async_copy(val_hbm.at[row], val_v, sem_v)
            cp_i.wait()
            cp_v.wait()

            @pl.loop(0, L_IN, step=LANES)
            def _(i):
                iv = idx_v[pl.ds(i, LANES)]
                vv = val_v[pl.ds(i, LANES)]
                plsc.store_scatter(buf_a, [iv], vv, mask=iv < HALF)

            pltpu.async_copy(buf_a, out_hbm.at[row, pl.ds(0, HALF)], sem_oa)

            @pl.loop(0, L_IN, step=LANES)
            def _(i):
                iv = idx_v[pl.ds(i, LANES)]
                vv = val_v[pl.ds(i, LANES)]
                m = (iv >= HALF) & (iv < L_OUT)
                plsc.store_scatter(
                    buf_b, [jnp.where(m, iv - HALF, 0)], vv, mask=m)

            pltpu.async_copy(buf_b, out_hbm.at[row, pl.ds(HALF, HALF)], sem_ob)

        last = base + ROWS_PER_W - 1
        pltpu.make_async_copy(
            buf_a, out_hbm.at[last, pl.ds(0, HALF)], sem_oa).wait()
        pltpu.make_async_copy(
            buf_b, out_hbm.at[last, pl.ds(HALF, HALF)], sem_ob).wait()

    return sc_kernel(vals, scat)


def kernel(input, indices):
    x = input.reshape(ROWS, L_IN)
    idx = indices.reshape(ROWS, L_IN).astype(jnp.int32)
    g = idx + jnp.arange(ROWS, dtype=jnp.int32)[:, None] * L_OUT
    ks, vs = lax.sort_key_val(g.reshape(-1), x.reshape(-1), is_stable=False)
    last = jnp.concatenate([ks[1:] != ks[:-1], jnp.ones((1,), jnp.bool_)])
    scat = jnp.where(last, ks % L_OUT, DUMP).astype(jnp.int32)
    out = _sc_unpool(vs.reshape(ROWS, L_IN), scat.reshape(ROWS, L_IN))
    return out.reshape(N, C, D_OUT, H_OUT, W_OUT)


# speedup vs baseline: 3.9814x; 1.0041x over previous
import dataclasses
import functools

import jax
import jax.numpy as jnp
from jax import lax
from jax.experimental import pallas as pl
from jax.experimental.pallas import tpu as pltpu
from jax.experimental.pallas import tpu_sc as plsc

N, C = 4, 64
D_IN, H_IN, W_IN = 16, 28, 28
D_OUT, H_OUT, W_OUT = 32, 56, 56
L_IN = D_IN * H_IN * W_IN
L_OUT = D_OUT * H_OUT * W_OUT
HALF = L_OUT // 2
ROWS = N * C
NUM_CORES = 2
NUM_SUBCORES = 16
NUM_WORKERS = NUM_CORES * NUM_SUBCORES
ROWS_PER_W = ROWS // NUM_WORKERS
LANES = 16


def _sc_unpool(vals, keys):
    mesh = plsc.VectorSubcoreMesh(core_axis_name="c", subcore_axis_name="s")
    cp = pltpu.CompilerParams()
    if "needs_layout_passes" in pltpu.CompilerParams.__dataclass_fields__:
        cp = dataclasses.replace(cp, needs_layout_passes=False)

    @functools.partial(
        pl.kernel,
        compiler_params=cp,
        out_type=jax.ShapeDtypeStruct((ROWS, L_OUT), jnp.float32),
        mesh=mesh,
        scratch_types=[
            pltpu.VMEM((HALF,), jnp.float32),
            pltpu.VMEM((HALF,), jnp.float32),
            pltpu.VMEM((L_IN,), jnp.int32),
            pltpu.VMEM((L_IN,), jnp.float32),
            pltpu.SemaphoreType.DMA,
            pltpu.SemaphoreType.DMA,
            pltpu.SemaphoreType.DMA,
            pltpu.SemaphoreType.DMA,
        ],
    )
    def sc_kernel(val_hbm, key_hbm, out_hbm, buf_a, buf_b, idx_v, val_v,
                  sem_i, sem_v, sem_oa, sem_ob):
        wid = lax.axis_index("s") * NUM_CORES + lax.axis_index("c")
        base = wid * ROWS_PER_W
        zeros = jnp.zeros((LANES,), jnp.float32)

        @pl.loop(0, HALF, step=LANES)
        def _(i):
            buf_a[pl.ds(i, LANES)] = zeros
            buf_b[pl.ds(i, LANES)] = zeros

        @pl.loop(0, ROWS_PER_W)
        def _(k):
            row = base + k

            @pl.when(k > 0)
            def _():
                prev = row - 1
                pbase = prev * L_OUT
                pltpu.make_async_copy(
                    buf_b, out_hbm.at[prev, pl.ds(HALF, HALF)], sem_ob
                ).wait()

                @pl.loop(0, L_IN, step=LANES)
                def _(i):
                    iv = idx_v[pl.ds(i, LANES)] - pbase
                    m = iv >= HALF
                    plsc.store_scatter(
                        buf_b, [jnp.where(m, iv - HALF, 0)], zeros, mask=m)

                pltpu.make_async_copy(
                    buf_a, out_hbm.at[prev, pl.ds(0, HALF)], sem_oa
                ).wait()

                @pl.loop(0, L_IN, step=LANES)
                def _(i):
                    iv = idx_v[pl.ds(i, LANES)] - pbase
                    m = iv < HALF
                    plsc.store_scatter(buf_a, [iv], zeros, mask=m)

            cp_i = pltpu.async_copy(key_hbm.at[row], idx_v, sem_i)
            cp_v = pltpu.async_copy(val_hbm.at[row], val_v, sem_v)
            cp_i.wait()
            cp_v.wait()
            rbase = row * L_OUT

            @pl.loop(0, L_IN, step=LANES)
            def _(i):
                iv = idx_v[pl.ds(i, LANES)] - rbase
                vv = val_v[pl.ds(i, LANES)]
                plsc.store_scatter(buf_a, [iv], vv, mask=iv < HALF)

            pltpu.async_copy(buf_a, out_hbm.at[row, pl.ds(0, HALF)], sem_oa)

            @pl.loop(0, L_IN, step=LANES)
            def _(i):
                iv = idx_v[pl.ds(i, LANES)] - rbase
                vv = val_v[pl.ds(i, LANES)]
                m = iv >= HALF
                plsc.store_scatter(
                    buf_b, [jnp.where(m, iv - HALF, 0)], vv, mask=m)

            pltpu.async_copy(buf_b, out_hbm.at[row, pl.ds(HALF, HALF)], sem_ob)

        last = base + ROWS_PER_W - 1
        pltpu.make_async_copy(
            buf_a, out_hbm.at[last, pl.ds(0, HALF)], sem_oa).wait()
        pltpu.make_async_copy(
            buf_b, out_hbm.at[last, pl.ds(HALF, HALF)], sem_ob).wait()

    return sc_kernel(vals, keys)


def kernel(input, indices):
    x = input.reshape(ROWS, L_IN)
    idx = indices.reshape(ROWS, L_IN).astype(jnp.int32)
    g = idx + jnp.arange(ROWS, dtype=jnp.int32)[:, None] * L_OUT
    ks, vs = lax.sort_key_val(g.reshape(-1), x.reshape(-1), is_stable=False)
    out = _sc_unpool(vs.reshape(ROWS, L_IN), ks.reshape(ROWS, L_IN))
    return out.reshape(N, C, D_OUT, H_OUT, W_OUT)
